# baseline (speedup 1.0000x reference)
import jax
import jax.numpy as jnp
from jax import lax
from jax.experimental import pallas as pl
from jax.experimental.pallas import tpu as pltpu
from jax.experimental.pallas import tpu_sc as plsc

N = 10000
E = 320000
D = 128
G = 64

NC = 2
NS = 16
NW = NC * NS

CB = 128
CHT = 80
NCHUNK = NW * CHT
EPAD = NCHUNK * CB - E
NPAD = 10240
RPT = NPAD // NS

_MESH = plsc.VectorSubcoreMesh(
    core_axis_name="c", subcore_axis_name="s", num_cores=NC, num_subcores=NS)


def _fill_rows(ref, nrows, value):
    vec = jnp.full((16,), value, jnp.float32)

    @pl.loop(0, nrows)
    def _(r):
        for j in range(D // 16):
            ref[r, pl.ds(j * 16, 16)] = vec


def _set_identity_idx(idx_v, start):
    iota16 = lax.broadcasted_iota(jnp.int32, (16,), 0)
    for j in range(CB // 16):
        idx_v[pl.ds(j * 16, 16)] = start + j * 16 + iota16


def _sc_agg_body(gather_ones,
                 table_hbm, src_hbm, dst2_hbm, agg_hbm,
                 src4, dst_all, rows2, idx_v, idx2_v, acc_s,
                 gsem0, gsem1, ssem0, ssem1, sem):
    cid = lax.axis_index("c")
    sid = lax.axis_index("s")
    wid = cid * NS + sid
    rb = sid * RPT

    gsems = (gsem0, gsem1)
    ssems = (ssem0, ssem1)
    rows = (rows2.at[0], rows2.at[1])

    c0 = wid * CHT

    pltpu.sync_copy(dst2_hbm.at[pl.ds(c0, CHT)], dst_all)

    _fill_rows(rows2.at[0], CB, 0.0)

    @pl.loop(0, RPT // CB)
    def _(k):
        _set_identity_idx(idx_v, rb + k * CB)
        pltpu.sync_copy(rows2.at[0], acc_s.at[idx_v])

    if gather_ones:
        _fill_rows(rows2.at[0], CB, 1.0)
        _fill_rows(rows2.at[1], CB, 1.0)

    plsc.subcore_barrier()

    def load_src(i, j):
        if not gather_ones:
            pltpu.sync_copy(src_hbm.at[pl.ds((c0 + i) * CB, CB)], src4.at[j])

    def gather_start(j, b):
        if not gather_ones:
            pltpu.async_copy(table_hbm.at[src4.at[j]], rows[b], gsems[b])

    def gather_drain(b):
        if not gather_ones:
            pltpu.make_async_copy(table_hbm.at[src4.at[0]], rows[b],
                                  gsems[b]).wait()

    load_src(0, 0)
    load_src(1, 1)
    gather_start(0, 0)
    gather_start(1, 1)

    @pl.loop(0, (CHT - 4) // 4)
    def _(k):
        i0 = 4 * k
        for b in range(4):
            rs = b % 2
            gather_drain(rs)
            cp = pltpu.async_copy(rows[rs], acc_s.at[dst_all.at[i0 + b]],
                                  ssems[rs], add=True)
            load_src(i0 + b + 2, (b + 2) % 4)
            cp.wait()
            gather_start((b + 2) % 4, rs)

    for b in (0, 1):
        i = CHT - 4 + b
        rs = b % 2
        gather_drain(rs)
        cp = pltpu.async_copy(rows[rs], acc_s.at[dst_all.at[i]],
                              ssems[rs], add=True)
        load_src(i + 2, (b + 2) % 4)
        cp.wait()
        gather_start((b + 2) % 4, rs)

    for i in (CHT - 2, CHT - 1):
        rs = i % 2
        gather_drain(rs)
        pltpu.async_copy(rows[rs], acc_s.at[dst_all.at[i]],
                         ssems[rs], add=True).wait()

    plsc.subcore_barrier()

    idxs = (idx_v, idx2_v)
    _set_identity_idx(idxs[0], rb)
    pltpu.async_copy(acc_s.at[idxs[0]], rows2.at[0], gsems[0])
    for k in range(RPT // CB):
        b = k % 2
        pltpu.make_async_copy(acc_s.at[idxs[b]], rows2.at[b], gsems[b]).wait()
        if k + 1 < RPT // CB:
            nb = (k + 1) % 2
            _set_identity_idx(idxs[nb], rb + (k + 1) * CB)
            pltpu.async_copy(acc_s.at[idxs[nb]], rows2.at[nb], gsems[nb])
        pltpu.sync_copy(rows2.at[b], agg_hbm.at[cid].at[pl.ds(rb + k * CB, CB)])


def _sc_aggregate(table, src, dst, gather_ones=False):
    def body(*refs):
        _sc_agg_body(gather_ones, *refs)

    k = pl.kernel(
        body,
        out_type=jax.ShapeDtypeStruct((NC, NPAD, D), jnp.float32),
        mesh=_MESH,
        scratch_types=[
            pltpu.VMEM((4, CB), jnp.int32),
            pltpu.VMEM((CHT, CB), jnp.int32),
            pltpu.VMEM((2, CB, D), jnp.float32),
            pltpu.VMEM((CB,), jnp.int32),
            pltpu.VMEM((CB,), jnp.int32),
            pltpu.VMEM_SHARED((NPAD, D), jnp.float32),
            pltpu.SemaphoreType.DMA,
            pltpu.SemaphoreType.DMA,
            pltpu.SemaphoreType.DMA,
            pltpu.SemaphoreType.DMA,
            pltpu.SemaphoreType.DMA,
        ],
        name="sc_segment_count" if gather_ones else "sc_segment_sum",
    )
    return k(table, src, dst)


def _sc_count(dst):
    dummy_table = jnp.zeros((8, D), jnp.float32)
    dummy_src = jnp.zeros((8,), jnp.int32)
    return _sc_aggregate(dummy_table, dummy_src, dst, gather_ones=True)


def _xw_block(x_ref, w_ref, b_ref, o_ref):
    o_ref[...] = (jnp.dot(x_ref[...], w_ref[...],
                          preferred_element_type=jnp.float32) + b_ref[...])


def _l1_block(aggp_ref, cntp_ref, xr_ref, wl_ref, o_ref):
    agg = aggp_ref[0] + aggp_ref[1]
    cnt = cntp_ref[0, :, 0:1] + cntp_ref[1, :, 0:1]
    mean = agg / jnp.maximum(cnt, 1.0)
    h = (jnp.dot(mean, wl_ref[...], preferred_element_type=jnp.float32)
         + xr_ref[...])
    o_ref[...] = jnp.maximum(h, 0.0)


def _l2_block(aggp_ref, cntp_ref, hr_ref, wl_ref, b_ref, o_ref):
    i = pl.program_id(0)
    agg = aggp_ref[0] + aggp_ref[1]
    cnt = cntp_ref[0, :, 0:1] + cntp_ref[1, :, 0:1]
    mean = agg / jnp.maximum(cnt, 1.0)
    h2 = (jnp.dot(mean, wl_ref[...], preferred_element_type=jnp.float32)
          + hr_ref[...])
    gids = b_ref[0, 0, :]
    gi = lax.broadcasted_iota(jnp.int32, (G, gids.shape[0]), 0)
    onehot = (gi == gids[None, :]).astype(jnp.float32)
    part = jnp.dot(onehot, h2, preferred_element_type=jnp.float32)

    @pl.when(i == 0)
    def _():
        o_ref[...] = part

    @pl.when(i != 0)
    def _():
        o_ref[...] += part


R = 2000
NB = N // R


def _tc_xw(x, w, b):
    return pl.pallas_call(
        _xw_block,
        grid=(NB,),
        in_specs=[
            pl.BlockSpec((R, D), lambda i: (i, 0)),
            pl.BlockSpec((D, D), lambda i: (0, 0)),
            pl.BlockSpec((1, D), lambda i: (0, 0)),
        ],
        out_specs=pl.BlockSpec((R, D), lambda i: (i, 0)),
        out_shape=jax.ShapeDtypeStruct((N, D), jnp.float32),
    )(x, w, b.reshape(1, D))


def kernel(x, edge_index, batch, W1l, b1l, W1r, W2l, b2l, W2r):
    src = jnp.concatenate(
        [edge_index[0], jnp.zeros((EPAD,), edge_index.dtype)])
    dst = jnp.concatenate(
        [edge_index[1], jnp.full((EPAD,), N, edge_index.dtype)])

    dst2 = dst.reshape(NCHUNK, CB)
    cntp = _sc_count(dst2)
    agg1p = _sc_aggregate(x, src, dst2)
    xr = _tc_xw(x, W1r, b1l)

    h = pl.pallas_call(
        _l1_block,
        grid=(NB,),
        in_specs=[
            pl.BlockSpec((NC, R, D), lambda i: (0, i, 0)),
            pl.BlockSpec((NC, R, D), lambda i: (0, i, 0)),
            pl.BlockSpec((R, D), lambda i: (i, 0)),
            pl.BlockSpec((D, D), lambda i: (0, 0)),
        ],
        out_specs=pl.BlockSpec((R, D), lambda i: (i, 0)),
        out_shape=jax.ShapeDtypeStruct((N, D), jnp.float32),
    )(agg1p, cntp, xr, W1l)

    agg2p = _sc_aggregate(h, src, dst2)
    hr = _tc_xw(h, W2r, b2l)

    out = pl.pallas_call(
        _l2_block,
        grid=(NB,),
        in_specs=[
            pl.BlockSpec((NC, R, D), lambda i: (0, i, 0)),
            pl.BlockSpec((NC, R, D), lambda i: (0, i, 0)),
            pl.BlockSpec((R, D), lambda i: (i, 0)),
            pl.BlockSpec((D, D), lambda i: (0, 0)),
            pl.BlockSpec((1, 1, R), lambda i: (i, 0, 0)),
        ],
        out_specs=pl.BlockSpec((G, D), lambda i: (0, 0)),
        out_shape=jax.ShapeDtypeStruct((G, D), jnp.float32),
    )(agg2p, cntp, hr, W2l, batch.reshape(NB, 1, R))

    return out


# speedup vs baseline: 1.0003x; 1.0003x over previous
import jax
import jax.numpy as jnp
from jax import lax
from jax.experimental import pallas as pl
from jax.experimental.pallas import tpu as pltpu
from jax.experimental.pallas import tpu_sc as plsc

N = 10000
E = 320000
D = 128
G = 64

NC = 2
NS = 16
NW = NC * NS

CB = 128
CHT = 80
NCHUNK = NW * CHT
EPAD = NCHUNK * CB - E
NPAD = 10240
RPT = NPAD // NS

_MESH = plsc.VectorSubcoreMesh(
    core_axis_name="c", subcore_axis_name="s", num_cores=NC, num_subcores=NS)


def _fill_rows(ref, nrows, value):
    vec = jnp.full((16,), value, jnp.float32)

    @pl.loop(0, nrows)
    def _(r):
        for j in range(D // 16):
            ref[r, pl.ds(j * 16, 16)] = vec


def _set_identity_idx(idx_v, start):
    iota16 = lax.broadcasted_iota(jnp.int32, (16,), 0)
    for j in range(CB // 16):
        idx_v[pl.ds(j * 16, 16)] = start + j * 16 + iota16


def _sc_agg_body(gather_ones,
                 table_hbm, src_hbm, dst2_hbm, agg_hbm,
                 src4, dst_all, rows2, idx_v, idx2_v, acc_s,
                 gsem0, gsem1, ssem0, ssem1, sem):
    cid = lax.axis_index("c")
    sid = lax.axis_index("s")
    wid = cid * NS + sid
    rb = sid * RPT

    gsems = (gsem0, gsem1)
    ssems = (ssem0, ssem1)
    rows = (rows2.at[0], rows2.at[1])

    c0 = wid * CHT

    pltpu.sync_copy(dst2_hbm.at[pl.ds(c0, CHT)], dst_all)

    _fill_rows(rows2.at[0], CB, 0.0)

    @pl.loop(0, RPT // CB)
    def _(k):
        _set_identity_idx(idx_v, rb + k * CB)
        pltpu.sync_copy(rows2.at[0], acc_s.at[idx_v])

    if gather_ones:
        _fill_rows(rows2.at[0], CB, 1.0)
        _fill_rows(rows2.at[1], CB, 1.0)

    plsc.subcore_barrier()

    def load_src(i, j):
        if not gather_ones:
            pltpu.sync_copy(src_hbm.at[pl.ds((c0 + i) * CB, CB)], src4.at[j])

    def gather_start(j, b):
        if not gather_ones:
            pltpu.async_copy(table_hbm.at[src4.at[j]], rows[b], gsems[b])

    def gather_drain(b):
        if not gather_ones:
            pltpu.make_async_copy(table_hbm.at[src4.at[0]], rows[b],
                                  gsems[b]).wait()

    load_src(0, 0)
    load_src(1, 1)
    gather_start(0, 0)
    gather_start(1, 1)

    @pl.loop(0, (CHT - 4) // 4)
    def _(k):
        i0 = 4 * k
        for b in range(4):
            rs = b % 2
            gather_drain(rs)
            cp = pltpu.async_copy(rows[rs], acc_s.at[dst_all.at[i0 + b]],
                                  ssems[rs], add=True)
            load_src(i0 + b + 2, (b + 2) % 4)
            cp.wait()
            gather_start((b + 2) % 4, rs)

    for b in (0, 1):
        i = CHT - 4 + b
        rs = b % 2
        gather_drain(rs)
        cp = pltpu.async_copy(rows[rs], acc_s.at[dst_all.at[i]],
                              ssems[rs], add=True)
        load_src(i + 2, (b + 2) % 4)
        cp.wait()
        gather_start((b + 2) % 4, rs)

    for i in (CHT - 2, CHT - 1):
        rs = i % 2
        gather_drain(rs)
        pltpu.async_copy(rows[rs], acc_s.at[dst_all.at[i]],
                         ssems[rs], add=True).wait()

    plsc.subcore_barrier()

    idxs = (idx_v, idx2_v)
    _set_identity_idx(idxs[0], rb)
    pltpu.async_copy(acc_s.at[idxs[0]], rows2.at[0], gsems[0])
    for k in range(RPT // CB):
        b = k % 2
        pltpu.make_async_copy(acc_s.at[idxs[b]], rows2.at[b], gsems[b]).wait()
        if k + 1 < RPT // CB:
            nb = (k + 1) % 2
            _set_identity_idx(idxs[nb], rb + (k + 1) * CB)
            pltpu.async_copy(acc_s.at[idxs[nb]], rows2.at[nb], gsems[nb])
        pltpu.sync_copy(rows2.at[b], agg_hbm.at[cid].at[pl.ds(rb + k * CB, CB)])


def _sc_aggregate(table, src, dst, gather_ones=False):
    def body(*refs):
        _sc_agg_body(gather_ones, *refs)

    k = pl.kernel(
        body,
        out_type=jax.ShapeDtypeStruct((NC, NPAD, D), jnp.float32),
        mesh=_MESH,
        scratch_types=[
            pltpu.VMEM((4, CB), jnp.int32),
            pltpu.VMEM((CHT, CB), jnp.int32),
            pltpu.VMEM((2, CB, D), jnp.float32),
            pltpu.VMEM((CB,), jnp.int32),
            pltpu.VMEM((CB,), jnp.int32),
            pltpu.VMEM_SHARED((NPAD, D), jnp.float32),
            pltpu.SemaphoreType.DMA,
            pltpu.SemaphoreType.DMA,
            pltpu.SemaphoreType.DMA,
            pltpu.SemaphoreType.DMA,
            pltpu.SemaphoreType.DMA,
        ],
        name="sc_segment_count" if gather_ones else "sc_segment_sum",
    )
    return k(table, src, dst)


def _sc_count(dst):
    dummy_table = jnp.zeros((8, D), jnp.float32)
    dummy_src = jnp.zeros((8,), jnp.int32)
    return _sc_aggregate(dummy_table, dummy_src, dst, gather_ones=True)


def _xw_block(x_ref, w_ref, b_ref, o_ref):
    o_ref[...] = (jnp.dot(x_ref[...], w_ref[...],
                          preferred_element_type=jnp.float32) + b_ref[...])


def _l1_block(aggp_ref, cntp_ref, xr_ref, wl_ref, o_ref):
    agg = aggp_ref[0] + aggp_ref[1]
    cnt = cntp_ref[0, :, 0:1] + cntp_ref[1, :, 0:1]
    mean = agg / jnp.maximum(cnt, 1.0)
    h = (jnp.dot(mean, wl_ref[...], preferred_element_type=jnp.float32)
         + xr_ref[...])
    o_ref[...] = jnp.maximum(h, 0.0)


def _l2_block(aggp_ref, cntp_ref, hr_ref, wl_ref, b_ref, o_ref):
    i = pl.program_id(0)
    agg = aggp_ref[0] + aggp_ref[1]
    cnt = cntp_ref[0, :, 0:1] + cntp_ref[1, :, 0:1]
    mean = agg / jnp.maximum(cnt, 1.0)
    h2 = (jnp.dot(mean, wl_ref[...], preferred_element_type=jnp.float32)
          + hr_ref[...])
    gids = b_ref[0, 0, :]
    gi = lax.broadcasted_iota(jnp.int32, (G, gids.shape[0]), 0)
    onehot = (gi == gids[None, :]).astype(jnp.float32)
    part = jnp.dot(onehot, h2, preferred_element_type=jnp.float32)

    @pl.when(i == 0)
    def _():
        o_ref[...] = part

    @pl.when(i != 0)
    def _():
        o_ref[...] += part


R = 2000
NB = N // R


def _tc_xw(x, w, b):
    return pl.pallas_call(
        _xw_block,
        grid=(NB,),
        in_specs=[
            pl.BlockSpec((R, D), lambda i: (i, 0)),
            pl.BlockSpec((D, D), lambda i: (0, 0)),
            pl.BlockSpec((1, D), lambda i: (0, 0)),
        ],
        out_specs=pl.BlockSpec((R, D), lambda i: (i, 0)),
        out_shape=jax.ShapeDtypeStruct((N, D), jnp.float32),
    )(x, w, b.reshape(1, D))


def kernel(x, edge_index, batch, W1l, b1l, W1r, W2l, b2l, W2r):
    src = jnp.concatenate(
        [edge_index[0], jnp.zeros((EPAD,), edge_index.dtype)])
    pad_dst = (N + jnp.arange(EPAD, dtype=edge_index.dtype) % (NPAD - N))
    dst = jnp.concatenate([edge_index[1], pad_dst])

    dst2 = dst.reshape(NCHUNK, CB)
    cntp = _sc_count(dst2)
    agg1p = _sc_aggregate(x, src, dst2)
    xr = _tc_xw(x, W1r, b1l)

    h = pl.pallas_call(
        _l1_block,
        grid=(NB,),
        in_specs=[
            pl.BlockSpec((NC, R, D), lambda i: (0, i, 0)),
            pl.BlockSpec((NC, R, D), lambda i: (0, i, 0)),
            pl.BlockSpec((R, D), lambda i: (i, 0)),
            pl.BlockSpec((D, D), lambda i: (0, 0)),
        ],
        out_specs=pl.BlockSpec((R, D), lambda i: (i, 0)),
        out_shape=jax.ShapeDtypeStruct((N, D), jnp.float32),
    )(agg1p, cntp, xr, W1l)

    agg2p = _sc_aggregate(h, src, dst2)
    hr = _tc_xw(h, W2r, b2l)

    out = pl.pallas_call(
        _l2_block,
        grid=(NB,),
        in_specs=[
            pl.BlockSpec((NC, R, D), lambda i: (0, i, 0)),
            pl.BlockSpec((NC, R, D), lambda i: (0, i, 0)),
            pl.BlockSpec((R, D), lambda i: (i, 0)),
            pl.BlockSpec((D, D), lambda i: (0, 0)),
            pl.BlockSpec((1, 1, R), lambda i: (i, 0, 0)),
        ],
        out_specs=pl.BlockSpec((G, D), lambda i: (0, 0)),
        out_shape=jax.ShapeDtypeStruct((G, D), jnp.float32),
    )(agg2p, cntp, hr, W2l, batch.reshape(NB, 1, R))

    return out


# speedup vs baseline: 3.0195x; 3.0186x over previous
import jax
import jax.numpy as jnp
from jax import lax
from jax.experimental import pallas as pl
from jax.experimental.pallas import tpu as pltpu
from jax.experimental.pallas import tpu_sc as plsc

N = 10000
E = 320000
D = 128
G = 64

NC = 2
NS = 16
NW = NC * NS

EPT = E // NW
CB = 128
NFULL = EPT // CB
TB = EPT - NFULL * CB
NPAD = 10240
RPT = NPAD // NS

_MESH = plsc.VectorSubcoreMesh(
    core_axis_name="c", subcore_axis_name="s", num_cores=NC, num_subcores=NS)


def _fill_rows(ref, nrows, value):
    vec = jnp.full((16,), value, jnp.float32)

    @pl.loop(0, nrows)
    def _(r):
        for j in range(D // 16):
            ref[r, pl.ds(j * 16, 16)] = vec


def _set_identity_idx(idx_v, start):
    iota16 = lax.broadcasted_iota(jnp.int32, (16,), 0)
    for j in range(CB // 16):
        idx_v[pl.ds(j * 16, 16)] = start + j * 16 + iota16


def _sc_agg_body(gather_ones,
                 table_hbm, src_hbm, dst_hbm, agg_hbm,
                 src4, dst4, rows2, src_t, dst_t, rows_t,
                 idx_v, idx2_v, acc_s, gsem0, gsem1, ssem0, ssem1, sem):
    cid = lax.axis_index("c")
    sid = lax.axis_index("s")
    wid = cid * NS + sid
    rb = sid * RPT

    gsems = (gsem0, gsem1)
    ssems = (ssem0, ssem1)
    rows = (rows2.at[0], rows2.at[1])

    _fill_rows(rows2.at[0], CB, 0.0)

    @pl.loop(0, RPT // CB)
    def _(k):
        _set_identity_idx(idx_v, rb + k * CB)
        pltpu.sync_copy(rows2.at[0], acc_s.at[idx_v])

    if gather_ones:
        _fill_rows(rows2.at[0], CB, 1.0)
        _fill_rows(rows2.at[1], CB, 1.0)
        _fill_rows(rows_t, TB, 1.0)

    plsc.subcore_barrier()

    base = wid * EPT

    def load_idx(c, j):
        pltpu.sync_copy(dst_hbm.at[pl.ds(base + c * CB, CB)], dst4.at[j])
        if not gather_ones:
            pltpu.sync_copy(src_hbm.at[pl.ds(base + c * CB, CB)], src4.at[j])

    def gather_start(j, b):
        if not gather_ones:
            pltpu.async_copy(table_hbm.at[src4.at[j]], rows[b], gsems[b])

    def gather_drain(b):
        if not gather_ones:
            pltpu.make_async_copy(table_hbm.at[src4.at[0]], rows[b],
                                  gsems[b]).wait()

    load_idx(0, 0)
    load_idx(1, 1)
    gather_start(0, 0)
    gather_start(1, 1)

    @pl.loop(0, NFULL // 4)
    def _(k):
        c0 = 4 * k
        for b in range(4):
            rs = b % 2
            gather_drain(rs)
            cp = pltpu.async_copy(rows[rs], acc_s.at[dst4.at[b]],
                                  ssems[rs], add=True)
            load_idx(c0 + b + 2, (b + 2) % 4)
            cp.wait()
            gather_start((b + 2) % 4, rs)

    for c, b in ((NFULL - 2, (NFULL - 2) % 4), (NFULL - 1, (NFULL - 1) % 4)):
        rs = b % 2
        gather_drain(rs)
        pltpu.async_copy(rows[rs], acc_s.at[dst4.at[b]],
                         ssems[rs], add=True).wait()

    toff = base + NFULL * CB
    pltpu.sync_copy(dst_hbm.at[pl.ds(toff, TB)], dst_t)
    if not gather_ones:
        pltpu.sync_copy(src_hbm.at[pl.ds(toff, TB)], src_t)
        pltpu.async_copy(table_hbm.at[src_t], rows_t, sem).wait()
    pltpu.sync_copy(rows_t, acc_s.at[dst_t], add=True)

    plsc.subcore_barrier()

    idxs = (idx_v, idx2_v)
    _set_identity_idx(idxs[0], rb)
    pltpu.async_copy(acc_s.at[idxs[0]], rows2.at[0], gsems[0])
    for k in range(RPT // CB):
        b = k % 2
        pltpu.make_async_copy(acc_s.at[idxs[b]], rows2.at[b], gsems[b]).wait()
        if k + 1 < RPT // CB:
            nb = (k + 1) % 2
            _set_identity_idx(idxs[nb], rb + (k + 1) * CB)
            pltpu.async_copy(acc_s.at[idxs[nb]], rows2.at[nb], gsems[nb])
        pltpu.sync_copy(rows2.at[b], agg_hbm.at[cid].at[pl.ds(rb + k * CB, CB)])


def _sc_aggregate(table, src, dst, gather_ones=False):
    def body(*refs):
        _sc_agg_body(gather_ones, *refs)

    k = pl.kernel(
        body,
        out_type=jax.ShapeDtypeStruct((NC, NPAD, D), jnp.float32),
        mesh=_MESH,
        scratch_types=[
            pltpu.VMEM((4, CB), jnp.int32),
            pltpu.VMEM((4, CB), jnp.int32),
            pltpu.VMEM((2, CB, D), jnp.float32),
            pltpu.VMEM((TB,), jnp.int32),
            pltpu.VMEM((TB,), jnp.int32),
            pltpu.VMEM((TB, D), jnp.float32),
            pltpu.VMEM((CB,), jnp.int32),
            pltpu.VMEM((CB,), jnp.int32),
            pltpu.VMEM_SHARED((NPAD, D), jnp.float32),
            pltpu.SemaphoreType.DMA,
            pltpu.SemaphoreType.DMA,
            pltpu.SemaphoreType.DMA,
            pltpu.SemaphoreType.DMA,
            pltpu.SemaphoreType.DMA,
        ],
        name="sc_segment_count" if gather_ones else "sc_segment_sum",
    )
    return k(table, src, dst)


def _sc_count(dst):
    dummy_table = jnp.zeros((8, D), jnp.float32)
    dummy_src = jnp.zeros((8,), jnp.int32)
    return _sc_aggregate(dummy_table, dummy_src, dst, gather_ones=True)


def _xw_block(x_ref, w_ref, b_ref, o_ref):
    o_ref[...] = (jnp.dot(x_ref[...], w_ref[...],
                          preferred_element_type=jnp.float32) + b_ref[...])


def _l1_block(aggp_ref, cntp_ref, xr_ref, wl_ref, o_ref):
    agg = aggp_ref[0] + aggp_ref[1]
    cnt = cntp_ref[0, :, 0:1] + cntp_ref[1, :, 0:1]
    mean = agg / jnp.maximum(cnt, 1.0)
    h = (jnp.dot(mean, wl_ref[...], preferred_element_type=jnp.float32)
         + xr_ref[...])
    o_ref[...] = jnp.maximum(h, 0.0)


def _l2_block(aggp_ref, cntp_ref, hr_ref, wl_ref, b_ref, o_ref):
    i = pl.program_id(0)
    agg = aggp_ref[0] + aggp_ref[1]
    cnt = cntp_ref[0, :, 0:1] + cntp_ref[1, :, 0:1]
    mean = agg / jnp.maximum(cnt, 1.0)
    h2 = (jnp.dot(mean, wl_ref[...], preferred_element_type=jnp.float32)
          + hr_ref[...])
    gids = b_ref[0, 0, :]
    gi = lax.broadcasted_iota(jnp.int32, (G, gids.shape[0]), 0)
    onehot = (gi == gids[None, :]).astype(jnp.float32)
    part = jnp.dot(onehot, h2, preferred_element_type=jnp.float32)

    @pl.when(i == 0)
    def _():
        o_ref[...] = part

    @pl.when(i != 0)
    def _():
        o_ref[...] += part


R = 2000
NB = N // R


def _tc_xw(x, w, b):
    return pl.pallas_call(
        _xw_block,
        grid=(NB,),
        in_specs=[
            pl.BlockSpec((R, D), lambda i: (i, 0)),
            pl.BlockSpec((D, D), lambda i: (0, 0)),
            pl.BlockSpec((1, D), lambda i: (0, 0)),
        ],
        out_specs=pl.BlockSpec((R, D), lambda i: (i, 0)),
        out_shape=jax.ShapeDtypeStruct((N, D), jnp.float32),
    )(x, w, b.reshape(1, D))


def kernel(x, edge_index, batch, W1l, b1l, W1r, W2l, b2l, W2r):
    src = edge_index[0]
    dst = edge_index[1]

    cntp = _sc_count(dst)
    agg1p = _sc_aggregate(x, src, dst)
    xr = _tc_xw(x, W1r, b1l)

    h = pl.pallas_call(
        _l1_block,
        grid=(NB,),
        in_specs=[
            pl.BlockSpec((NC, R, D), lambda i: (0, i, 0)),
            pl.BlockSpec((NC, R, D), lambda i: (0, i, 0)),
            pl.BlockSpec((R, D), lambda i: (i, 0)),
            pl.BlockSpec((D, D), lambda i: (0, 0)),
        ],
        out_specs=pl.BlockSpec((R, D), lambda i: (i, 0)),
        out_shape=jax.ShapeDtypeStruct((N, D), jnp.float32),
    )(agg1p, cntp, xr, W1l)

    agg2p = _sc_aggregate(h, src, dst)
    hr = _tc_xw(h, W2r, b2l)

    out = pl.pallas_call(
        _l2_block,
        grid=(NB,),
        in_specs=[
            pl.BlockSpec((NC, R, D), lambda i: (0, i, 0)),
            pl.BlockSpec((NC, R, D), lambda i: (0, i, 0)),
            pl.BlockSpec((R, D), lambda i: (i, 0)),
            pl.BlockSpec((D, D), lambda i: (0, 0)),
            pl.BlockSpec((1, 1, R), lambda i: (i, 0, 0)),
        ],
        out_specs=pl.BlockSpec((G, D), lambda i: (0, 0)),
        out_shape=jax.ShapeDtypeStruct((G, D), jnp.float32),
    )(agg2p, cntp, hr, W2l, batch.reshape(NB, 1, R))

    return out


# speedup vs baseline: 3.0323x; 1.0042x over previous
import jax
import jax.numpy as jnp
from jax import lax
from jax.experimental import pallas as pl
from jax.experimental.pallas import tpu as pltpu
from jax.experimental.pallas import tpu_sc as plsc

N = 10000
E = 320000
D = 128
G = 64

NC = 2
NS = 16
NW = NC * NS

EPT = E // NW
CB = 128
NFULL = EPT // CB
TB = EPT - NFULL * CB
NPAD = 10240
RPT = NPAD // NS

_MESH = plsc.VectorSubcoreMesh(
    core_axis_name="c", subcore_axis_name="s", num_cores=NC, num_subcores=NS)


def _fill_rows(ref, nrows, value):
    vec = jnp.full((16,), value, jnp.float32)

    @pl.loop(0, nrows)
    def _(r):
        for j in range(D // 16):
            ref[r, pl.ds(j * 16, 16)] = vec


def _set_identity_idx(idx_v, start):
    iota16 = lax.broadcasted_iota(jnp.int32, (16,), 0)
    for j in range(CB // 16):
        idx_v[pl.ds(j * 16, 16)] = start + j * 16 + iota16


def _sc_agg_body(gather_ones,
                 table_hbm, src_hbm, dst_hbm, agg_hbm,
                 src4, dst4, rows2, src_t, dst_t, rows_t,
                 idx_v, idx2_v, acc_s, gsem0, gsem1, ssem0, ssem1, sem):
    cid = lax.axis_index("c")
    sid = lax.axis_index("s")
    wid = cid * NS + sid
    rb = sid * RPT

    gsems = (gsem0, gsem1)
    ssems = (ssem0, ssem1)
    rows = (rows2.at[0], rows2.at[1])

    _fill_rows(rows2.at[0], CB, 0.0)

    @pl.loop(0, RPT // CB)
    def _(k):
        _set_identity_idx(idx_v, rb + k * CB)
        pltpu.sync_copy(rows2.at[0], acc_s.at[idx_v])

    if gather_ones:
        _fill_rows(rows2.at[0], CB, 1.0)
        _fill_rows(rows2.at[1], CB, 1.0)
        _fill_rows(rows_t, TB, 1.0)

    plsc.subcore_barrier()

    base = wid * EPT

    def load_idx(c, j):
        pltpu.sync_copy(dst_hbm.at[pl.ds(base + c * CB, CB)], dst4.at[j])
        if not gather_ones:
            pltpu.sync_copy(src_hbm.at[pl.ds(base + c * CB, CB)], src4.at[j])

    def gather_start(j, b):
        if not gather_ones:
            pltpu.async_copy(table_hbm.at[src4.at[j]], rows[b], gsems[b])

    def gather_drain(b):
        if not gather_ones:
            pltpu.make_async_copy(table_hbm.at[src4.at[0]], rows[b],
                                  gsems[b]).wait()

    if not gather_ones:
        load_idx(0, 0)
        load_idx(1, 1)
        gather_start(0, 0)
        gather_start(1, 1)

        @pl.loop(0, NFULL // 4)
        def _(k):
            c0 = 4 * k
            for b in range(4):
                rs = b % 2
                gather_drain(rs)
                cp = pltpu.async_copy(rows[rs], acc_s.at[dst4.at[b]],
                                      ssems[rs], add=True)
                load_idx(c0 + b + 2, (b + 2) % 4)
                cp.wait()
                gather_start((b + 2) % 4, rs)

        for c, b in ((NFULL - 2, (NFULL - 2) % 4),
                     (NFULL - 1, (NFULL - 1) % 4)):
            rs = b % 2
            gather_drain(rs)
            pltpu.async_copy(rows[rs], acc_s.at[dst4.at[b]],
                             ssems[rs], add=True).wait()
    else:
        def wait_scat(s):
            pltpu.make_async_copy(rows[s], acc_s.at[dst4.at[0]],
                                  ssems[s]).wait()

        load_idx(0, 0)
        load_idx(1, 1)
        pltpu.async_copy(rows[0], acc_s.at[dst4.at[0]], ssems[0], add=True)
        load_idx(2, 2)

        @pl.loop(0, (NFULL - 2) // 2)
        def _(k):
            c = 2 * k + 1
            pltpu.async_copy(rows[1], acc_s.at[dst4.at[lax.rem(c, 4)]],
                             ssems[1], add=True)
            wait_scat(0)

            @pl.when(c + 2 < NFULL)
            def _():
                load_idx(c + 2, lax.rem(c + 2, 4))

            pltpu.async_copy(rows[0], acc_s.at[dst4.at[lax.rem(c + 1, 4)]],
                             ssems[0], add=True)
            wait_scat(1)

            @pl.when(c + 3 < NFULL)
            def _():
                load_idx(c + 3, lax.rem(c + 3, 4))

        pltpu.async_copy(rows[1], acc_s.at[dst4.at[(NFULL - 1) % 4]],
                         ssems[1], add=True)
        wait_scat(0)
        wait_scat(1)

    toff = base + NFULL * CB
    pltpu.sync_copy(dst_hbm.at[pl.ds(toff, TB)], dst_t)
    if not gather_ones:
        pltpu.sync_copy(src_hbm.at[pl.ds(toff, TB)], src_t)
        pltpu.async_copy(table_hbm.at[src_t], rows_t, sem).wait()
    pltpu.sync_copy(rows_t, acc_s.at[dst_t], add=True)

    plsc.subcore_barrier()

    idxs = (idx_v, idx2_v)
    _set_identity_idx(idxs[0], rb)
    pltpu.async_copy(acc_s.at[idxs[0]], rows2.at[0], gsems[0])
    for k in range(RPT // CB):
        b = k % 2
        pltpu.make_async_copy(acc_s.at[idxs[b]], rows2.at[b], gsems[b]).wait()
        if k + 1 < RPT // CB:
            nb = (k + 1) % 2
            _set_identity_idx(idxs[nb], rb + (k + 1) * CB)
            pltpu.async_copy(acc_s.at[idxs[nb]], rows2.at[nb], gsems[nb])
        pltpu.sync_copy(rows2.at[b], agg_hbm.at[cid].at[pl.ds(rb + k * CB, CB)])


def _sc_aggregate(table, src, dst, gather_ones=False):
    def body(*refs):
        _sc_agg_body(gather_ones, *refs)

    k = pl.kernel(
        body,
        out_type=jax.ShapeDtypeStruct((NC, NPAD, D), jnp.float32),
        mesh=_MESH,
        scratch_types=[
            pltpu.VMEM((4, CB), jnp.int32),
            pltpu.VMEM((4, CB), jnp.int32),
            pltpu.VMEM((2, CB, D), jnp.float32),
            pltpu.VMEM((TB,), jnp.int32),
            pltpu.VMEM((TB,), jnp.int32),
            pltpu.VMEM((TB, D), jnp.float32),
            pltpu.VMEM((CB,), jnp.int32),
            pltpu.VMEM((CB,), jnp.int32),
            pltpu.VMEM_SHARED((NPAD, D), jnp.float32),
            pltpu.SemaphoreType.DMA,
            pltpu.SemaphoreType.DMA,
            pltpu.SemaphoreType.DMA,
            pltpu.SemaphoreType.DMA,
            pltpu.SemaphoreType.DMA,
        ],
        name="sc_segment_count" if gather_ones else "sc_segment_sum",
    )
    return k(table, src, dst)


def _sc_count(dst):
    dummy_table = jnp.zeros((8, D), jnp.float32)
    dummy_src = jnp.zeros((8,), jnp.int32)
    return _sc_aggregate(dummy_table, dummy_src, dst, gather_ones=True)


def _xw_block(x_ref, w_ref, b_ref, o_ref):
    o_ref[...] = (jnp.dot(x_ref[...], w_ref[...],
                          preferred_element_type=jnp.float32) + b_ref[...])


def _l1_block(aggp_ref, cntp_ref, xr_ref, wl_ref, o_ref):
    agg = aggp_ref[0] + aggp_ref[1]
    cnt = cntp_ref[0, :, 0:1] + cntp_ref[1, :, 0:1]
    mean = agg / jnp.maximum(cnt, 1.0)
    h = (jnp.dot(mean, wl_ref[...], preferred_element_type=jnp.float32)
         + xr_ref[...])
    o_ref[...] = jnp.maximum(h, 0.0)


def _l2_block(aggp_ref, cntp_ref, hr_ref, wl_ref, b_ref, o_ref):
    i = pl.program_id(0)
    agg = aggp_ref[0] + aggp_ref[1]
    cnt = cntp_ref[0, :, 0:1] + cntp_ref[1, :, 0:1]
    mean = agg / jnp.maximum(cnt, 1.0)
    h2 = (jnp.dot(mean, wl_ref[...], preferred_element_type=jnp.float32)
          + hr_ref[...])
    gids = b_ref[0, 0, :]
    gi = lax.broadcasted_iota(jnp.int32, (G, gids.shape[0]), 0)
    onehot = (gi == gids[None, :]).astype(jnp.float32)
    part = jnp.dot(onehot, h2, preferred_element_type=jnp.float32)

    @pl.when(i == 0)
    def _():
        o_ref[...] = part

    @pl.when(i != 0)
    def _():
        o_ref[...] += part


R = 2000
NB = N // R


def _tc_xw(x, w, b):
    return pl.pallas_call(
        _xw_block,
        grid=(NB,),
        in_specs=[
            pl.BlockSpec((R, D), lambda i: (i, 0)),
            pl.BlockSpec((D, D), lambda i: (0, 0)),
            pl.BlockSpec((1, D), lambda i: (0, 0)),
        ],
        out_specs=pl.BlockSpec((R, D), lambda i: (i, 0)),
        out_shape=jax.ShapeDtypeStruct((N, D), jnp.float32),
    )(x, w, b.reshape(1, D))


def kernel(x, edge_index, batch, W1l, b1l, W1r, W2l, b2l, W2r):
    src = edge_index[0]
    dst = edge_index[1]

    cntp = _sc_count(dst)
    agg1p = _sc_aggregate(x, src, dst)
    xr = _tc_xw(x, W1r, b1l)

    h = pl.pallas_call(
        _l1_block,
        grid=(NB,),
        in_specs=[
            pl.BlockSpec((NC, R, D), lambda i: (0, i, 0)),
            pl.BlockSpec((NC, R, D), lambda i: (0, i, 0)),
            pl.BlockSpec((R, D), lambda i: (i, 0)),
            pl.BlockSpec((D, D), lambda i: (0, 0)),
        ],
        out_specs=pl.BlockSpec((R, D), lambda i: (i, 0)),
        out_shape=jax.ShapeDtypeStruct((N, D), jnp.float32),
    )(agg1p, cntp, xr, W1l)

    agg2p = _sc_aggregate(h, src, dst)
    hr = _tc_xw(h, W2r, b2l)

    out = pl.pallas_call(
        _l2_block,
        grid=(NB,),
        in_specs=[
            pl.BlockSpec((NC, R, D), lambda i: (0, i, 0)),
            pl.BlockSpec((NC, R, D), lambda i: (0, i, 0)),
            pl.BlockSpec((R, D), lambda i: (i, 0)),
            pl.BlockSpec((D, D), lambda i: (0, 0)),
            pl.BlockSpec((1, 1, R), lambda i: (i, 0, 0)),
        ],
        out_specs=pl.BlockSpec((G, D), lambda i: (0, 0)),
        out_shape=jax.ShapeDtypeStruct((G, D), jnp.float32),
    )(agg2p, cntp, hr, W2l, batch.reshape(NB, 1, R))

    return out
